# baseline (speedup 1.0000x reference)
import jax
import jax.numpy as jnp
from jax.experimental import pallas as pl
from jax.experimental.pallas import tpu as pltpu

_N, _T, _U, _FEAT = 8, 512, 64, 80
_D = 512
_V = 500
_S = 5
_NEG = -1.0e30
_F32 = jnp.float32
_U1 = _U + 1


def _bdot(a, b):
    return jnp.dot(a.astype(jnp.bfloat16), b.astype(jnp.bfloat16),
                   preferred_element_type=_F32)


def _lae(x, y):
    m = jnp.maximum(x, y)
    return m + jnp.log1p(jnp.exp(-jnp.abs(x - y)))


def _shr(x, d, fill):
    pad = jnp.full(x.shape[:-1] + (d,), fill, x.dtype)
    return jnp.concatenate([pad, x[..., :-d]], axis=-1)


def _mega_body(x_ref, y_ref, tl_ref, ul_ref, xl_ref, yl_ref,
               encW_ref, encb_ref, ctcW_ref, amW_ref, jencW_ref,
               sy_ref, ye_ref, emb_ref, decW_ref, lmW_ref, jdecW_ref,
               joW_ref,
               ctc_ref, simple_ref, pruned_ref,
               amp_sc, lm_sc, lmp_sc, yoh_sc, sk_sc):
    rows = _N * _U1
    iov = jax.lax.broadcasted_iota(jnp.int32, (rows, _V), 1)
    onehot = (iov == sy_ref[...]).astype(_F32)
    demb = _bdot(onehot, emb_ref[...])
    dec = jnp.tanh(_bdot(demb, decW_ref[...]))
    lm_all = _bdot(dec, lmW_ref[...])
    lmp_all = _bdot(dec, jdecW_ref[...])
    yoh_all = (iov == ye_ref[...]).astype(_F32)
    for bb in range(_N):
        lm_sc[bb, 0:_U1, :] = lm_all[bb * _U1:(bb + 1) * _U1]
        lmp_sc[bb, 0:_U1, :] = lmp_all[bb * _U1:(bb + 1) * _U1]
        yoh_sc[bb, 0:_U1, :] = yoh_all[bb * _U1:(bb + 1) * _U1]

    eye_u1 = (jax.lax.broadcasted_iota(jnp.int32, (_U1, _U1), 0)
              == jax.lax.broadcasted_iota(jnp.int32, (_U1, _U1), 1)
              ).astype(_F32)
    eye_u = (jax.lax.broadcasted_iota(jnp.int32, (_U, _U), 0)
             == jax.lax.broadcasted_iota(jnp.int32, (_U, _U), 1)).astype(_F32)
    amW16 = amW_ref[...].astype(jnp.bfloat16)
    jencW16 = jencW_ref[...].astype(jnp.bfloat16)
    for bb in range(_N):
        x = x_ref[bb]
        enc = jnp.tanh(
            jnp.dot(x, encW_ref[...], preferred_element_type=_F32)
            + encb_ref[...])
        ctc = jnp.dot(enc, ctcW_ref[...], preferred_element_type=_F32)
        m = jnp.max(ctc, axis=-1, keepdims=True)
        lse = jnp.log(jnp.sum(jnp.exp(ctc - m), axis=-1, keepdims=True)) + m
        ctc_ref[bb] = ctc - lse
        encb16 = enc.astype(jnp.bfloat16)
        am = jnp.dot(encb16, amW16, preferred_element_type=_F32)
        amp_sc[bb] = jnp.dot(encb16, jencW16, preferred_element_type=_F32)

        lm = lm_sc[bb, 0:_U1, :]
        amm = jnp.max(am, axis=-1, keepdims=True)
        lmm = jnp.max(lm, axis=-1, keepdims=True)
        ea = jnp.exp(am - amm)
        el = jnp.exp(lm - lmm)
        z = jax.lax.dot_general(ea, el, (((1,), (1,)), ((), ())),
                                preferred_element_type=_F32)
        lmm_row = jnp.sum(eye_u1 * lmm, axis=0, keepdims=True)
        norm = jnp.log(z) + amm + lmm_row

        y_row = y_ref[bb]
        oh_vu = (jax.lax.broadcasted_iota(jnp.int32, (_V, _U), 0)
                 == y_row).astype(_F32)
        am_y = jnp.dot(am, oh_vu, preferred_element_type=_F32)
        y_col = jnp.sum(eye_u * y_row.astype(_F32), axis=1, keepdims=True)
        oh_uv = (jax.lax.broadcasted_iota(jnp.int32, (_U, _V), 1)
                 == y_col.astype(jnp.int32)).astype(_F32)
        lm_y_col = jnp.sum(lm[:_U, :] * oh_uv, axis=1, keepdims=True)
        lm_y_row = jnp.sum(eye_u * lm_y_col, axis=0, keepdims=True)
        lmb_col = lm[:, 0:1]
        lmb_row = jnp.sum(eye_u1 * lmb_col, axis=0, keepdims=True)

        emit = am_y + lm_y_row - norm[:, :_U]
        blank = am[:, 0:1] + lmb_row - norm
        cat = jnp.concatenate(
            [emit, blank, jnp.full((_T, 256 - _U - _U1), _NEG, _F32)],
            axis=1)
        cat_t = jnp.transpose(cat)
        sk_sc[0:_U + _U1, bb, :] = cat_t[0:_U + _U1]

    lane_t = jax.lax.broadcasted_iota(jnp.int32, (_N, _T), 1)
    tl_oh = lane_t == tl_ref[...]
    ul = ul_ref[...]

    def pick(f, blc):
        v = jnp.where(tl_oh, f + blc, _NEG)
        return jnp.max(v, axis=1, keepdims=True)

    b0 = sk_sc[_U]
    c = _shr(b0, 1, 0.0)
    for d in (1, 2, 4, 8, 16, 32, 64, 128, 256):
        c = c + _shr(c, d, 0.0)
    fin0 = jnp.where(ul == 0, pick(c, b0), jnp.full((_N, 1), _NEG, _F32))

    def step(u, carry):
        f, fin = carry
        g = f + sk_sc[u - 1]
        blc = sk_sc[_U + u]
        a = _shr(blc, 1, _NEG)
        b = g
        for d in (1, 2, 4, 8, 16, 32, 64, 128, 256):
            a_sh = _shr(a, d, 0.0)
            b_sh = _shr(b, d, _NEG)
            b = _lae(b, a + b_sh)
            a = a + a_sh
        fin = jnp.where(ul == u, pick(b, blc), fin)
        return b, fin

    _, fin = jax.lax.fori_loop(1, _U1, step, (c, fin0))
    simple_ref[...] = -jnp.sum(fin, axis=0, keepdims=True)

    w16 = joW_ref[...].astype(jnp.bfloat16)
    tidx = jax.lax.broadcasted_iota(jnp.int32, (_T, 1), 0)
    acc = jnp.zeros((1, 1), _F32)
    for bb in range(_N):
        tlen = xl_ref[bb]
        ulen = yl_ref[bb]
        amp = amp_sc[bb]
        lmp16 = lmp_sc[bb, 0:_U1, :].astype(jnp.bfloat16)
        yoh16 = yoh_sc[bb, 0:_U1, :].astype(jnp.bfloat16)

        tmask = (tidx < tlen).astype(_F32)
        tf = tidx.astype(_F32)
        denom = jnp.maximum(tlen, 1).astype(_F32)
        center = jnp.floor(tf * ulen.astype(_F32) / denom).astype(jnp.int32)
        hi = jnp.maximum(ulen + 1 - _S, 0)
        start = jnp.clip(center - _S // 2, 0, hi)

        for s in range(_S):
            r_s = start + s
            roh16 = (jax.lax.broadcasted_iota(jnp.int32, (_T, _U1), 1)
                     == r_s).astype(jnp.bfloat16)
            lm_s = jnp.dot(roh16, lmp16, preferred_element_type=_F32)
            h16 = jnp.tanh(amp + lm_s).astype(jnp.bfloat16)
            logits = jnp.dot(h16, w16, preferred_element_type=_F32)
            m = jnp.max(logits, axis=-1, keepdims=True)
            lse = jnp.log(jnp.sum(jnp.exp(logits - m), axis=-1,
                                  keepdims=True)) + m
            symoh = jnp.dot(roh16, yoh16, preferred_element_type=_F32)
            emitv = jnp.sum(logits * symoh, axis=-1, keepdims=True)
            blankv = logits[:, 0:1]
            umask = (r_s < ulen).astype(_F32)
            contrib = tmask * (umask * (emitv - lse) + (blankv - lse))
            acc = acc + jnp.sum(contrib, axis=0, keepdims=True)

    pruned_ref[...] = -acc / _S


def kernel(x, x_lens, y_padded, y_lens, enc_W, enc_b, dec_emb, dec_W,
           join_enc_W, join_dec_W, join_out_W, simple_am_W, simple_lm_W,
           ctc_W):
    x_lens = x_lens.astype(jnp.int32)
    y_lens = y_lens.astype(jnp.int32)
    y_padded = y_padded.astype(jnp.int32)

    sos_y = jnp.concatenate(
        [jnp.zeros((_N, 1), jnp.int32), y_padded], axis=1).reshape(-1, 1)
    y_ext = jnp.concatenate(
        [y_padded, jnp.zeros((_N, 1), jnp.int32)], axis=1).reshape(-1, 1)

    in_specs = [
        pl.BlockSpec((_N, _T, _FEAT), lambda: (0, 0, 0)),
        pl.BlockSpec((_N, 1, _U), lambda: (0, 0, 0)),
        pl.BlockSpec((_N, 1), lambda: (0, 0)),
        pl.BlockSpec((_N, 1), lambda: (0, 0)),
        pl.BlockSpec(memory_space=pltpu.SMEM),
        pl.BlockSpec(memory_space=pltpu.SMEM),
        pl.BlockSpec((_FEAT, _D), lambda: (0, 0)),
        pl.BlockSpec((1, _D), lambda: (0, 0)),
        pl.BlockSpec((_D, _V), lambda: (0, 0)),
        pl.BlockSpec((_D, _V), lambda: (0, 0)),
        pl.BlockSpec((_D, _D), lambda: (0, 0)),
        pl.BlockSpec((_N * _U1, 1), lambda: (0, 0)),
        pl.BlockSpec((_N * _U1, 1), lambda: (0, 0)),
        pl.BlockSpec((_V, _D), lambda: (0, 0)),
        pl.BlockSpec((_D, _D), lambda: (0, 0)),
        pl.BlockSpec((_D, _V), lambda: (0, 0)),
        pl.BlockSpec((_D, _D), lambda: (0, 0)),
        pl.BlockSpec((_D, _V), lambda: (0, 0)),
    ]
    ctc_out, simple, pruned = pl.pallas_call(
        _mega_body,
        in_specs=in_specs,
        out_shape=[
            jax.ShapeDtypeStruct((_N, _T, _V), _F32),
            jax.ShapeDtypeStruct((1, 1), _F32),
            jax.ShapeDtypeStruct((1, 1), _F32),
        ],
        scratch_shapes=[
            pltpu.VMEM((_N, _T, _D), _F32),
            pltpu.VMEM((_N, _U1 + 7, _V), _F32),
            pltpu.VMEM((_N, _U1 + 7, _D), _F32),
            pltpu.VMEM((_N, _U1 + 7, _V), _F32),
            pltpu.VMEM((_U + _U1, _N, _T), _F32),
        ],
    )(x, y_padded.reshape(_N, 1, _U),
      (x_lens - 1).reshape(_N, 1), y_lens.reshape(_N, 1),
      x_lens, y_lens,
      enc_W, enc_b.reshape(1, _D), ctc_W, simple_am_W, join_enc_W,
      sos_y, y_ext, dec_emb, dec_W, simple_lm_W, join_dec_W,
      join_out_W)

    return simple.reshape(()), pruned.reshape(()), ctc_out


# speedup vs baseline: 1.0148x; 1.0148x over previous
import jax
import jax.numpy as jnp
from jax.experimental import pallas as pl
from jax.experimental.pallas import tpu as pltpu

_N, _T, _U, _FEAT = 8, 512, 64, 80
_D = 512
_V = 500
_S = 5
_NEG = -1.0e30
_F32 = jnp.float32
_U1 = _U + 1


def _lae(x, y):
    m = jnp.maximum(x, y)
    return m + jnp.log1p(jnp.exp(-jnp.abs(x - y)))


_SP = (0.6938791410346656, 0.5067399580390645, 0.13938461132799188,
       0.011392770258370318, -0.0021592553883983687,
       -0.0005147263066368846, -3.027787176981847e-05)


def _lae_fast(x, y):
    m = jnp.maximum(x, y)
    z = jnp.maximum(-jnp.abs(x - y), -6.0)
    p = jnp.float32(_SP[6])
    for c in _SP[5::-1]:
        p = jnp.float32(c) + z * p
    return m + p


def _shr(x, d, fill):
    pad = jnp.full(x.shape[:-1] + (d,), fill, x.dtype)
    return jnp.concatenate([pad, x[..., :-d]], axis=-1)


def _mega_body(x_ref, y_ref, tl_ref, ul_ref, xl_ref, yl_ref,
               encW_ref, encb_ref, ctcW_ref, amW_ref, jencW_ref,
               sy_ref, ye_ref, emb_ref, decW_ref, lmW_ref, jdecW_ref,
               joW_ref,
               ctc_ref, simple_ref, pruned_ref,
               amp_sc, lm_sc, lmp_sc, yoh_sc, sk_sc, a_sc):
    rows = _N * _U1
    iov = jax.lax.broadcasted_iota(jnp.int32, (rows, _V), 1)
    onehot = (iov == sy_ref[...]).astype(_F32)
    demb = jnp.dot(onehot, emb_ref[...], preferred_element_type=_F32)
    dec = jnp.tanh(jnp.dot(demb, decW_ref[...], preferred_element_type=_F32))
    lm_all = jnp.dot(dec, lmW_ref[...], preferred_element_type=_F32)
    lmp_all = jnp.dot(dec, jdecW_ref[...], preferred_element_type=_F32)
    yoh_all = (iov == ye_ref[...]).astype(_F32)
    for bb in range(_N):
        lm_sc[bb, 0:_U1, :] = lm_all[bb * _U1:(bb + 1) * _U1]
        lmp_sc[bb, 0:_U1, :] = lmp_all[bb * _U1:(bb + 1) * _U1]
        yoh_sc[bb, 0:_U1, :] = yoh_all[bb * _U1:(bb + 1) * _U1]

    eye_u1 = (jax.lax.broadcasted_iota(jnp.int32, (_U1, _U1), 0)
              == jax.lax.broadcasted_iota(jnp.int32, (_U1, _U1), 1)
              ).astype(_F32)
    eye_u = (jax.lax.broadcasted_iota(jnp.int32, (_U, _U), 0)
             == jax.lax.broadcasted_iota(jnp.int32, (_U, _U), 1)).astype(_F32)
    for bb in range(_N):
        x = x_ref[bb]
        enc = jnp.tanh(
            jnp.dot(x, encW_ref[...], preferred_element_type=_F32)
            + encb_ref[...])
        ctc = jnp.dot(enc, ctcW_ref[...], preferred_element_type=_F32)
        m = jnp.max(ctc, axis=-1, keepdims=True)
        lse = jnp.log(jnp.sum(jnp.exp(ctc - m), axis=-1, keepdims=True)) + m
        ctc_ref[bb] = ctc - lse
        am = jnp.dot(enc, amW_ref[...], preferred_element_type=_F32)
        amp_sc[bb] = jnp.dot(enc, jencW_ref[...], preferred_element_type=_F32)

        lm = lm_sc[bb, 0:_U1, :]
        amm = jnp.max(am, axis=-1, keepdims=True)
        lmm = jnp.max(lm, axis=-1, keepdims=True)
        ea = jnp.exp(am - amm)
        el = jnp.exp(lm - lmm)
        z = jax.lax.dot_general(ea, el, (((1,), (1,)), ((), ())),
                                preferred_element_type=_F32)
        lmm_row = jnp.sum(eye_u1 * lmm, axis=0, keepdims=True)
        norm = jnp.log(z) + amm + lmm_row

        y_row = y_ref[bb]
        oh_vu = (jax.lax.broadcasted_iota(jnp.int32, (_V, _U), 0)
                 == y_row).astype(_F32)
        am_y = jnp.dot(am, oh_vu, preferred_element_type=_F32)
        y_col = jnp.sum(eye_u * y_row.astype(_F32), axis=1, keepdims=True)
        oh_uv = (jax.lax.broadcasted_iota(jnp.int32, (_U, _V), 1)
                 == y_col.astype(jnp.int32)).astype(_F32)
        lm_y_col = jnp.sum(lm[:_U, :] * oh_uv, axis=1, keepdims=True)
        lm_y_row = jnp.sum(eye_u * lm_y_col, axis=0, keepdims=True)
        lmb_col = lm[:, 0:1]
        lmb_row = jnp.sum(eye_u1 * lmb_col, axis=0, keepdims=True)

        emit = am_y + lm_y_row - norm[:, :_U]
        blank = am[:, 0:1] + lmb_row - norm
        cat = jnp.concatenate(
            [emit, blank, jnp.full((_T, 256 - _U - _U1), _NEG, _F32)],
            axis=1)
        cat_t = jnp.transpose(cat)
        sk_sc[0:_U + _U1, bb, :] = cat_t[0:_U + _U1]

    lev = _shr(sk_sc[_U1:_U + _U1], 1, _NEG)
    a_sc[0:_U] = lev
    for k, d in enumerate((1, 2, 4, 8, 16, 32, 64, 128)):
        lev = lev + _shr(lev, d, 0.0)
        a_sc[_U * (k + 1):_U * (k + 2)] = lev

    lane_t = jax.lax.broadcasted_iota(jnp.int32, (_N, _T), 1)
    tl_oh = lane_t == tl_ref[...]
    ul = ul_ref[...]

    def pick(f, blc):
        v = jnp.where(tl_oh, f + blc, _NEG)
        return jnp.max(v, axis=1, keepdims=True)

    b0 = sk_sc[_U]
    c = _shr(b0, 1, 0.0)
    for d in (1, 2, 4, 8, 16, 32, 64, 128, 256):
        c = c + _shr(c, d, 0.0)
    fin0 = jnp.where(ul == 0, pick(c, b0), jnp.full((_N, 1), _NEG, _F32))

    def step(u, carry):
        f, fin = carry
        b = f + sk_sc[u - 1]
        blc = sk_sc[_U + u]
        for i, dd in enumerate((1, 8, 64)):
            base = 3 * i * _U + u - 1
            a0 = a_sc[base]
            a1 = a_sc[base + _U]
            a2 = a_sc[base + 2 * _U]
            c3 = a1 + _shr(a0, 2 * dd, 0.0)
            c5 = a2 + _shr(a0, 4 * dd, 0.0)
            c6 = a2 + _shr(a1, 4 * dd, 0.0)
            c7 = c6 + _shr(a0, 6 * dd, 0.0)
            t0 = b
            t1 = a0 + _shr(b, dd, _NEG)
            t2 = a1 + _shr(b, 2 * dd, _NEG)
            t3 = c3 + _shr(b, 3 * dd, _NEG)
            t4 = a2 + _shr(b, 4 * dd, _NEG)
            t5 = c5 + _shr(b, 5 * dd, _NEG)
            t6 = c6 + _shr(b, 6 * dd, _NEG)
            t7 = c7 + _shr(b, 7 * dd, _NEG)
            b = _lae_fast(
                _lae_fast(_lae_fast(t0, t1), _lae_fast(t2, t3)),
                _lae_fast(_lae_fast(t4, t5), _lae_fast(t6, t7)))
        fin = jnp.where(ul == u, pick(b, blc), fin)
        return b, fin

    _, fin = jax.lax.fori_loop(1, _U1, step, (c, fin0))
    simple_ref[...] = -jnp.sum(fin, axis=0, keepdims=True)

    w = joW_ref[...]
    tidx = jax.lax.broadcasted_iota(jnp.int32, (_T, 1), 0)
    acc = jnp.zeros((1, 1), _F32)
    for bb in range(_N):
        tlen = xl_ref[bb]
        ulen = yl_ref[bb]
        amp = amp_sc[bb]
        lmp = lmp_sc[bb, 0:_U1, :]
        yoh = yoh_sc[bb, 0:_U1, :]

        tmask = (tidx < tlen).astype(_F32)
        tf = tidx.astype(_F32)
        denom = jnp.maximum(tlen, 1).astype(_F32)
        center = jnp.floor(tf * ulen.astype(_F32) / denom).astype(jnp.int32)
        hi = jnp.maximum(ulen + 1 - _S, 0)
        start = jnp.clip(center - _S // 2, 0, hi)

        for s in range(_S):
            r_s = start + s
            roh = (jax.lax.broadcasted_iota(jnp.int32, (_T, _U1), 1)
                   == r_s).astype(_F32)
            lm_s = jnp.dot(roh, lmp, preferred_element_type=_F32)
            h = jnp.tanh(amp + lm_s)
            logits = jnp.dot(h, w, preferred_element_type=_F32)
            m = jnp.max(logits, axis=-1, keepdims=True)
            lse = jnp.log(jnp.sum(jnp.exp(logits - m), axis=-1,
                                  keepdims=True)) + m
            symoh = jnp.dot(roh, yoh, preferred_element_type=_F32)
            emitv = jnp.sum(logits * symoh, axis=-1, keepdims=True)
            blankv = logits[:, 0:1]
            umask = (r_s < ulen).astype(_F32)
            contrib = tmask * (umask * (emitv - lse) + (blankv - lse))
            acc = acc + jnp.sum(contrib, axis=0, keepdims=True)

    pruned_ref[...] = -acc / _S


def kernel(x, x_lens, y_padded, y_lens, enc_W, enc_b, dec_emb, dec_W,
           join_enc_W, join_dec_W, join_out_W, simple_am_W, simple_lm_W,
           ctc_W):
    x_lens = x_lens.astype(jnp.int32)
    y_lens = y_lens.astype(jnp.int32)
    y_padded = y_padded.astype(jnp.int32)

    sos_y = jnp.concatenate(
        [jnp.zeros((_N, 1), jnp.int32), y_padded], axis=1).reshape(-1, 1)
    y_ext = jnp.concatenate(
        [y_padded, jnp.zeros((_N, 1), jnp.int32)], axis=1).reshape(-1, 1)

    in_specs = [
        pl.BlockSpec((_N, _T, _FEAT), lambda: (0, 0, 0)),
        pl.BlockSpec((_N, 1, _U), lambda: (0, 0, 0)),
        pl.BlockSpec((_N, 1), lambda: (0, 0)),
        pl.BlockSpec((_N, 1), lambda: (0, 0)),
        pl.BlockSpec(memory_space=pltpu.SMEM),
        pl.BlockSpec(memory_space=pltpu.SMEM),
        pl.BlockSpec((_FEAT, _D), lambda: (0, 0)),
        pl.BlockSpec((1, _D), lambda: (0, 0)),
        pl.BlockSpec((_D, _V), lambda: (0, 0)),
        pl.BlockSpec((_D, _V), lambda: (0, 0)),
        pl.BlockSpec((_D, _D), lambda: (0, 0)),
        pl.BlockSpec((_N * _U1, 1), lambda: (0, 0)),
        pl.BlockSpec((_N * _U1, 1), lambda: (0, 0)),
        pl.BlockSpec((_V, _D), lambda: (0, 0)),
        pl.BlockSpec((_D, _D), lambda: (0, 0)),
        pl.BlockSpec((_D, _V), lambda: (0, 0)),
        pl.BlockSpec((_D, _D), lambda: (0, 0)),
        pl.BlockSpec((_D, _V), lambda: (0, 0)),
    ]
    ctc_out, simple, pruned = pl.pallas_call(
        _mega_body,
        in_specs=in_specs,
        out_shape=[
            jax.ShapeDtypeStruct((_N, _T, _V), _F32),
            jax.ShapeDtypeStruct((1, 1), _F32),
            jax.ShapeDtypeStruct((1, 1), _F32),
        ],
        scratch_shapes=[
            pltpu.VMEM((_N, _T, _D), _F32),
            pltpu.VMEM((_N, _U1 + 7, _V), _F32),
            pltpu.VMEM((_N, _U1 + 7, _D), _F32),
            pltpu.VMEM((_N, _U1 + 7, _V), _F32),
            pltpu.VMEM((_U + _U1, _N, _T), _F32),
            pltpu.VMEM((9 * _U, _N, _T), _F32),
        ],
    )(x, y_padded.reshape(_N, 1, _U),
      (x_lens - 1).reshape(_N, 1), y_lens.reshape(_N, 1),
      x_lens, y_lens,
      enc_W, enc_b.reshape(1, _D), ctc_W, simple_am_W, join_enc_W,
      sos_y, y_ext, dec_emb, dec_W, simple_lm_W, join_dec_W,
      join_out_W)

    return simple.reshape(()), pruned.reshape(()), ctc_out
